# Initial kernel scaffold; baseline (speedup 1.0000x reference)
import jax
import jax.numpy as jnp
from jax.experimental import pallas as pl


def kernel(x, y, Wg1, W1a, W1b, Wg2, W2a, W2b):
    raise NotImplementedError("write your pallas kernel here")


# speedup vs baseline: 1.5944x; 1.5944x over previous
import functools

import jax
import jax.numpy as jnp
from jax import lax
from jax.experimental import pallas as pl
from jax.experimental.pallas import tpu as pltpu

_T = 2048
_D = 1024
_F = 1024
_E = 8


def _router_body(x_ref, wg_ref, idx_ref, gv_ref, gates_ref):
    x = x_ref[...]
    wg = wg_ref[...]
    logits = lax.dot_general(
        x, wg, (((1,), (0,)), ((), ())),
        preferred_element_type=jnp.float32,
        precision=lax.Precision.HIGHEST,
    )
    m = jnp.max(logits, axis=1, keepdims=True)
    p = jnp.exp(logits - m)
    p = p / jnp.sum(p, axis=1, keepdims=True)
    lane = lax.broadcasted_iota(jnp.int32, p.shape, 1)
    m1 = jnp.max(p, axis=1, keepdims=True)
    i1 = jnp.min(jnp.where(p == m1, lane, _E), axis=1, keepdims=True)
    p2 = jnp.where(lane == i1, -1.0, p)
    m2 = jnp.max(p2, axis=1, keepdims=True)
    i2 = jnp.min(jnp.where(p2 == m2, lane, _E), axis=1, keepdims=True)
    den = m1 + m2 + 1e-9
    g1 = m1 / den
    g2 = m2 / den
    idx_ref[...] = jnp.concatenate([i1, i2], axis=1)
    gv_ref[...] = jnp.concatenate([g1, g2], axis=1)
    gates_ref[...] = jnp.where(lane == i1, g1, 0.0) + jnp.where(lane == i2, g2, 0.0)


def _router(xt, wg):
    return pl.pallas_call(
        _router_body,
        out_shape=(
            jax.ShapeDtypeStruct((_T, 2), jnp.int32),
            jax.ShapeDtypeStruct((_T, 2), jnp.float32),
            jax.ShapeDtypeStruct((_T, _E), jnp.float32),
        ),
    )(xt, wg)


def _dense_moe_body(gates_ref, x_ref, w1_ref, w2_ref, out_ref):
    e = pl.program_id(1)
    x = x_ref[...].astype(jnp.bfloat16)
    gates = gates_ref[...]
    lane = lax.broadcasted_iota(jnp.int32, gates.shape, 1)
    w1 = w1_ref[0].astype(jnp.bfloat16)
    w2 = w2_ref[0].astype(jnp.bfloat16)
    h = lax.dot_general(
        x, w1, (((1,), (0,)), ((), ())),
        preferred_element_type=jnp.float32,
    )
    h = jnp.maximum(h, 0.0).astype(jnp.bfloat16)
    y = lax.dot_general(
        h, w2, (((1,), (0,)), ((), ())),
        preferred_element_type=jnp.float32,
    )
    g = jnp.sum(jnp.where(lane == e, gates, 0.0), axis=1, keepdims=True)
    contrib = g * y

    @pl.when(e == 0)
    def _():
        out_ref[...] = contrib

    @pl.when(e > 0)
    def _():
        out_ref[...] = out_ref[...] + contrib


def _dense_moe(gates, xt, w1, w2, bt=1024):
    nt = _T // bt
    return pl.pallas_call(
        _dense_moe_body,
        grid=(nt, _E),
        in_specs=[
            pl.BlockSpec((bt, _E), lambda t, e: (t, 0)),
            pl.BlockSpec((bt, _D), lambda t, e: (t, 0)),
            pl.BlockSpec((1, _D, _F), lambda t, e: (e, 0, 0)),
            pl.BlockSpec((1, _F, _D), lambda t, e: (e, 0, 0)),
        ],
        out_specs=pl.BlockSpec((bt, _D), lambda t, e: (t, 0)),
        out_shape=jax.ShapeDtypeStruct((_T, _D), jnp.float32),
    )(gates, xt, w1, w2)


def _loss_body(y_ref, x_ref, m_ref, out_ref, acc_ref):
    t = pl.program_id(0)
    blk = x_ref[...] + m_ref[...]
    s = jnp.sum(blk, axis=0, keepdims=True)

    @pl.when(t == 0)
    def _():
        acc_ref[...] = s

    @pl.when(t > 0)
    def _():
        acc_ref[...] = acc_ref[...] + s

    @pl.when(t == pl.num_programs(0) - 1)
    def _():
        sent = acc_ref[...] / float(_T)
        mx = jnp.max(sent)
        lse = jnp.log(jnp.sum(jnp.exp(sent - mx))) + mx
        yv = y_ref[0]
        lane = lax.broadcasted_iota(jnp.int32, sent.shape, 1)
        picked = jnp.sum(jnp.where(lane == yv, sent, 0.0))
        out_ref[0, 0] = lse - picked


def _loss(y, xt, moe_out, bt=512):
    nt = _T // bt
    return pl.pallas_call(
        _loss_body,
        grid=(nt,),
        in_specs=[
            pl.BlockSpec(memory_space=pltpu.SMEM),
            pl.BlockSpec((bt, _D), lambda t: (t, 0)),
            pl.BlockSpec((bt, _D), lambda t: (t, 0)),
        ],
        out_specs=pl.BlockSpec(memory_space=pltpu.SMEM),
        out_shape=jax.ShapeDtypeStruct((1, 1), jnp.float32),
        scratch_shapes=[pltpu.VMEM((1, _D), jnp.float32)],
    )(y, xt, moe_out)


def kernel(x, y, Wg1, W1a, W1b, Wg2, W2a, W2b):
    xt = x.reshape(_T, _D)
    _, _, gates1 = _router(xt, Wg1)
    m1 = _dense_moe(gates1, xt, W1a, W1b)
    _, _, gates2 = _router(m1, Wg2)
    m2 = _dense_moe(gates2, m1, W2a, W2b)
    out = _loss(y.astype(jnp.int32), xt, m2)
    return out[0, 0]
